# baseline (speedup 1.0000x reference)
import functools

import jax
import jax.numpy as jnp
from jax import lax
from jax.experimental import pallas as pl
from jax.experimental.pallas import tpu as pltpu
from jax.experimental.pallas import tpu_sc as plsc

L = 16


@functools.lru_cache(maxsize=None)
def _build(B: int, N: int):
    info = plsc.get_sparse_core_info()
    NC, NS = info.num_cores, info.num_subcores
    NW = NC * NS
    b_per_w = B // NW
    CH = 8192
    n_chunks = b_per_w // CH
    mesh = plsc.VectorSubcoreMesh(core_axis_name="c", subcore_axis_name="s")

    @functools.partial(
        pl.kernel,
        mesh=mesh,
        out_type=jax.ShapeDtypeStruct((NW, L), jnp.float32),
        scratch_types=[
            pltpu.VMEM((b_per_w,), jnp.int32),
            pltpu.VMEM((b_per_w,), jnp.float32),
            pltpu.VMEM((CH,), jnp.float32),
            pltpu.VMEM((CH,), jnp.float32),
            pltpu.VMEM((CH,), jnp.float32),
            pltpu.VMEM((CH,), jnp.float32),
            pltpu.VMEM((L,), jnp.float32),
            pltpu.SemaphoreType.DMA,
            pltpu.SemaphoreType.DMA,
            pltpu.SemaphoreType.DMA,
            pltpu.SemaphoreType.DMA,
            pltpu.SemaphoreType.DMA,
            pltpu.SemaphoreType.DMA,
        ],
    )
    def k(x_hbm, idx_hbm, g_hbm, h_hbm, out_hbm,
          idx_v, x_v, g0, h0, g1, h1, acc_v,
          s_i, s_x, sg0, sh0, sg1, sh1):
        wid = lax.axis_index("s") * NC + lax.axis_index("c")
        base = wid * b_per_w
        bufs = ((g0, h0, sg0, sh0), (g1, h1, sg1, sh1))

        cp_i = pltpu.async_copy(idx_hbm.at[pl.ds(base, b_per_w)], idx_v, s_i)
        cp_xf = pltpu.async_copy(x_hbm.at[pl.ds(base, b_per_w)], x_v, s_x)
        cp_i.wait()

        def start_gathers(ci, b):
            g_v, h_v, sg, sh = bufs[b]
            idx_sl = idx_v.at[pl.ds(ci * CH, CH)]
            cp_g = pltpu.async_copy(g_hbm.at[idx_sl], g_v, sg)
            cp_h = pltpu.async_copy(h_hbm.at[idx_sl], h_v, sh)
            return cp_g, cp_h

        UNROLL = 4

        def make_body(xoff, g_v, h_v):
            def vec_body(i, accs):
                out = []
                for j in range(UNROLL):
                    off = (i * UNROLL + j) * L
                    xx = x_v[pl.ds(xoff + off, L)]
                    g = g_v[pl.ds(off, L)]
                    h = h_v[pl.ds(off, L)]
                    out.append(accs[j] + xx * (g + 0.5 * h * xx))
                return tuple(out)
            return vec_body

        inflight = [start_gathers(0, 0), None]
        cp_xf.wait()
        accs = tuple(jnp.zeros((L,), jnp.float32) for _ in range(UNROLL))
        for ci in range(n_chunks):
            b = ci & 1
            if ci + 1 < n_chunks:
                inflight[b ^ 1] = start_gathers(ci + 1, b ^ 1)
            cp_g, cp_h = inflight[b]
            cp_g.wait()
            cp_h.wait()
            g_v, h_v = bufs[b][0], bufs[b][1]
            accs = lax.fori_loop(0, CH // L // UNROLL,
                                 make_body(ci * CH, g_v, h_v), accs)

        acc_v[...] = accs[0] + accs[1] + (accs[2] + accs[3])
        pltpu.sync_copy(acc_v, out_hbm.at[wid])

    return k


def kernel(x, index, G_k, H_k):
    B = x.shape[0]
    N = G_k.shape[0]
    k = _build(B, N)
    partials = k(x, index.astype(jnp.int32), G_k, H_k)
    return jnp.sum(partials) * jnp.float32(1.0 / B)


# speedup vs baseline: 1.0046x; 1.0046x over previous
import functools

import jax
import jax.numpy as jnp
from jax import lax
from jax.experimental import pallas as pl
from jax.experimental.pallas import tpu as pltpu
from jax.experimental.pallas import tpu_sc as plsc

L = 16
NBUF = 4


@functools.lru_cache(maxsize=None)
def _build(B: int, N: int):
    info = plsc.get_sparse_core_info()
    NC, NS = info.num_cores, info.num_subcores
    NW = NC * NS
    b_per_w = B // NW
    CH = 4096
    n_chunks = b_per_w // CH
    mesh = plsc.VectorSubcoreMesh(core_axis_name="c", subcore_axis_name="s")

    gh_types = []
    for _ in range(NBUF):
        gh_types += [pltpu.VMEM((CH,), jnp.float32),
                     pltpu.VMEM((CH,), jnp.float32)]

    @functools.partial(
        pl.kernel,
        mesh=mesh,
        out_type=jax.ShapeDtypeStruct((NW, L), jnp.float32),
        scratch_types=[
            pltpu.VMEM((b_per_w,), jnp.int32),
            pltpu.VMEM((b_per_w,), jnp.float32),
        ] + gh_types + [
            pltpu.VMEM((L,), jnp.float32),
            pltpu.SemaphoreType.DMA,
            pltpu.SemaphoreType.DMA,
        ] + [pltpu.SemaphoreType.DMA] * (2 * NBUF),
    )
    def k(x_hbm, idx_hbm, g_hbm, h_hbm, out_hbm,
          idx_v, x_v, *rest):
        gh_bufs = rest[:2 * NBUF]
        acc_v = rest[2 * NBUF]
        s_i, s_x = rest[2 * NBUF + 1], rest[2 * NBUF + 2]
        gh_sems = rest[2 * NBUF + 3:]
        bufs = tuple(
            (gh_bufs[2 * b], gh_bufs[2 * b + 1],
             gh_sems[2 * b], gh_sems[2 * b + 1])
            for b in range(NBUF))

        wid = lax.axis_index("s") * NC + lax.axis_index("c")
        base = wid * b_per_w

        cp_i = pltpu.async_copy(idx_hbm.at[pl.ds(base, b_per_w)], idx_v, s_i)
        cp_xf = pltpu.async_copy(x_hbm.at[pl.ds(base, b_per_w)], x_v, s_x)
        cp_i.wait()

        def start_gathers(ci):
            g_v, h_v, sg, sh = bufs[ci % NBUF]
            idx_sl = idx_v.at[pl.ds(ci * CH, CH)]
            cp_g = pltpu.async_copy(g_hbm.at[idx_sl], g_v, sg)
            cp_h = pltpu.async_copy(h_hbm.at[idx_sl], h_v, sh)
            return cp_g, cp_h

        UNROLL = 4

        def make_body(xoff, g_v, h_v):
            def vec_body(i, accs):
                out = []
                for j in range(UNROLL):
                    off = (i * UNROLL + j) * L
                    xx = x_v[pl.ds(xoff + off, L)]
                    g = g_v[pl.ds(off, L)]
                    h = h_v[pl.ds(off, L)]
                    out.append(accs[j] + xx * (g + 0.5 * h * xx))
                return tuple(out)
            return vec_body

        inflight = [start_gathers(ci) for ci in range(min(NBUF, n_chunks))]
        cp_xf.wait()
        accs = tuple(jnp.zeros((L,), jnp.float32) for _ in range(UNROLL))
        for ci in range(n_chunks):
            cp_g, cp_h = inflight[ci % NBUF]
            cp_g.wait()
            cp_h.wait()
            g_v, h_v = bufs[ci % NBUF][0], bufs[ci % NBUF][1]
            accs = lax.fori_loop(0, CH // L // UNROLL,
                                 make_body(ci * CH, g_v, h_v), accs)
            if ci + NBUF < n_chunks:
                inflight[ci % NBUF] = start_gathers(ci + NBUF)

        acc_v[...] = accs[0] + accs[1] + (accs[2] + accs[3])
        pltpu.sync_copy(acc_v, out_hbm.at[wid])

    return k


def kernel(x, index, G_k, H_k):
    B = x.shape[0]
    N = G_k.shape[0]
    k = _build(B, N)
    partials = k(x, index.astype(jnp.int32), G_k, H_k)
    return jnp.sum(partials) * jnp.float32(1.0 / B)


# speedup vs baseline: 1.0086x; 1.0040x over previous
import functools

import jax
import jax.numpy as jnp
from jax import lax
from jax.experimental import pallas as pl
from jax.experimental.pallas import tpu as pltpu
from jax.experimental.pallas import tpu_sc as plsc

L = 16


@functools.lru_cache(maxsize=None)
def _build(B: int, N: int):
    info = plsc.get_sparse_core_info()
    NC, NS = info.num_cores, info.num_subcores
    NW = NC * NS
    b_per_w = B // NW
    CH = 8192
    n_chunks = b_per_w // CH
    mesh = plsc.VectorSubcoreMesh(core_axis_name="c", subcore_axis_name="s")

    buf_types = []
    for _ in range(2):
        buf_types += [
            pltpu.VMEM((CH,), jnp.int32),
            pltpu.VMEM((CH,), jnp.float32),
            pltpu.VMEM((CH,), jnp.float32),
            pltpu.VMEM((CH,), jnp.float32),
        ]
    sem_types = [pltpu.SemaphoreType.DMA] * 8

    @functools.partial(
        pl.kernel,
        mesh=mesh,
        out_type=jax.ShapeDtypeStruct((NW, L), jnp.float32),
        scratch_types=buf_types + [pltpu.VMEM((L,), jnp.float32)] + sem_types,
    )
    def k(x_hbm, idx_hbm, g_hbm, h_hbm, out_hbm,
          idx0, x0, g0, h0, idx1, x1, g1, h1, acc_v,
          si0, sx0, sg0, sh0, si1, sx1, sg1, sh1):
        wid = lax.axis_index("s") * NC + lax.axis_index("c")
        base = wid * b_per_w
        bufs = ((idx0, x0, g0, h0, si0, sx0, sg0, sh0),
                (idx1, x1, g1, h1, si1, sx1, sg1, sh1))

        def start_stage(ci, b):
            idx_v, x_v, g_v, h_v, si, sx, sg, sh = bufs[b]
            off = base + ci * CH
            cp_i = pltpu.async_copy(idx_hbm.at[pl.ds(off, CH)], idx_v, si)
            cp_x = pltpu.async_copy(x_hbm.at[pl.ds(off, CH)], x_v, sx)
            cp_i.wait()
            cp_g = pltpu.async_copy(g_hbm.at[idx_v], g_v, sg)
            cp_h = pltpu.async_copy(h_hbm.at[idx_v], h_v, sh)
            return cp_x, cp_g, cp_h

        UNROLL = 4

        def make_body(x_v, g_v, h_v):
            def vec_body(i, accs):
                out = []
                for j in range(UNROLL):
                    off = (i * UNROLL + j) * L
                    xx = x_v[pl.ds(off, L)]
                    g = g_v[pl.ds(off, L)]
                    h = h_v[pl.ds(off, L)]
                    out.append(accs[j] + xx * (g + 0.5 * h * xx))
                return tuple(out)
            return vec_body

        inflight = [start_stage(0, 0), None]
        accs = tuple(jnp.zeros((L,), jnp.float32) for _ in range(UNROLL))
        for ci in range(n_chunks):
            b = ci & 1
            if ci + 1 < n_chunks:
                inflight[b ^ 1] = start_stage(ci + 1, b ^ 1)
            cp_x, cp_g, cp_h = inflight[b]
            cp_x.wait()
            cp_g.wait()
            cp_h.wait()
            x_v, g_v, h_v = bufs[b][1], bufs[b][2], bufs[b][3]
            accs = lax.fori_loop(0, CH // L // UNROLL,
                                 make_body(x_v, g_v, h_v), accs)

        acc_v[...] = accs[0] + accs[1] + (accs[2] + accs[3])
        pltpu.sync_copy(acc_v, out_hbm.at[wid])

    return k


def kernel(x, index, G_k, H_k):
    B = x.shape[0]
    N = G_k.shape[0]
    k = _build(B, N)
    partials = k(x, index.astype(jnp.int32), G_k, H_k)
    return jnp.sum(partials) * jnp.float32(1.0 / B)


# speedup vs baseline: 1.0100x; 1.0013x over previous
import functools

import jax
import jax.numpy as jnp
from jax import lax
from jax.experimental import pallas as pl
from jax.experimental.pallas import tpu as pltpu
from jax.experimental.pallas import tpu_sc as plsc

L = 16


@functools.lru_cache(maxsize=None)
def _build(B: int, N: int):
    info = plsc.get_sparse_core_info()
    NC, NS = info.num_cores, info.num_subcores
    NW = NC * NS
    b_per_w = B // NW
    CH = 8192
    n_chunks = b_per_w // CH
    mesh = plsc.VectorSubcoreMesh(core_axis_name="c", subcore_axis_name="s")

    buf_types = []
    for _ in range(2):
        buf_types += [
            pltpu.VMEM((CH,), jnp.int32),
            pltpu.VMEM((CH,), jnp.float32),
            pltpu.VMEM((CH,), jnp.float32),
            pltpu.VMEM((CH,), jnp.float32),
        ]
    sem_types = [pltpu.SemaphoreType.DMA] * 8

    @functools.partial(
        pl.kernel,
        mesh=mesh,
        out_type=jax.ShapeDtypeStruct((NW * L,), jnp.float32),
        scratch_types=buf_types + [pltpu.VMEM((L,), jnp.float32)] + sem_types,
    )
    def k(x_hbm, idx_hbm, g_hbm, h_hbm, out_hbm,
          idx0, x0, g0, h0, idx1, x1, g1, h1, acc_v,
          si0, sx0, sg0, sh0, si1, sx1, sg1, sh1):
        wid = lax.axis_index("s") * NC + lax.axis_index("c")
        base = wid * b_per_w
        bufs = ((idx0, x0, g0, h0, si0, sx0, sg0, sh0),
                (idx1, x1, g1, h1, si1, sx1, sg1, sh1))

        def start_stage(ci, b):
            idx_v, x_v, g_v, h_v, si, sx, sg, sh = bufs[b]
            off = base + ci * CH
            cp_i = pltpu.async_copy(idx_hbm.at[pl.ds(off, CH)], idx_v, si)
            cp_x = pltpu.async_copy(x_hbm.at[pl.ds(off, CH)], x_v, sx)
            cp_i.wait()
            cp_g = pltpu.async_copy(g_hbm.at[idx_v], g_v, sg)
            cp_h = pltpu.async_copy(h_hbm.at[idx_v], h_v, sh)
            return cp_x, cp_g, cp_h

        UNROLL = 4

        def make_body(x_v, g_v, h_v):
            def vec_body(i, accs):
                out = []
                for j in range(UNROLL):
                    off = (i * UNROLL + j) * L
                    xx = x_v[pl.ds(off, L)]
                    g = g_v[pl.ds(off, L)]
                    h = h_v[pl.ds(off, L)]
                    out.append(accs[j] + xx * (g + 0.5 * h * xx))
                return tuple(out)
            return vec_body

        inflight = [start_stage(0, 0), None]
        accs = tuple(jnp.zeros((L,), jnp.float32) for _ in range(UNROLL))
        for ci in range(n_chunks):
            b = ci & 1
            if ci + 1 < n_chunks:
                inflight[b ^ 1] = start_stage(ci + 1, b ^ 1)
            cp_x, cp_g, cp_h = inflight[b]
            cp_x.wait()
            cp_g.wait()
            cp_h.wait()
            x_v, g_v, h_v = bufs[b][1], bufs[b][2], bufs[b][3]
            accs = lax.fori_loop(0, CH // L // UNROLL,
                                 make_body(x_v, g_v, h_v), accs)

        acc_v[...] = accs[0] + accs[1] + (accs[2] + accs[3])
        pltpu.sync_copy(acc_v, out_hbm.at[pl.ds(wid * L, L)])

    return k


def kernel(x, index, G_k, H_k):
    B = x.shape[0]
    N = G_k.shape[0]
    k = _build(B, N)
    partials = k(x, index.astype(jnp.int32), G_k, H_k)
    return jnp.sum(partials) * jnp.float32(1.0 / B)
